# baseline (speedup 1.0000x reference)
import functools

import jax
import jax.numpy as jnp
from jax import lax
from jax.experimental import pallas as pl
from jax.experimental.pallas import tpu as pltpu
from jax.experimental.pallas import tpu_sc as plsc

N = 10000
E = 160000
D = 128
DW = 144

NS = 16
CHUNK = 128
CHUNKS = 80
EPAD = NS * CHUNKS * CHUNK
ROWS_PER_SUB = 640
ACC_ROWS = NS * ROWS_PER_SUB
DUMMY = N
IDX_STAGE = 5


def _edge_prep(ei):
    pad = EPAD - E
    src = jnp.concatenate([ei[0], jnp.zeros((pad,), jnp.int32)])
    dst = jnp.concatenate([ei[1], jnp.full((pad,), DUMMY, jnp.int32)])
    return src.reshape(NS, CHUNKS, CHUNK), dst.reshape(NS, CHUNKS, CHUNK)


def _widen(x):
    return jnp.concatenate(
        [x, jnp.ones((N, 1), jnp.float32), jnp.zeros((N, DW - D - 1),
                                                     jnp.float32)], axis=1)


def _sc_aggregate_one(x_ref, sidx_ref, didx_ref, sum_ref,
                      sidx_v, didx_v, rows_a, rows_b,
                      acc_sh, gsem0, gsem1, srsem):
    sid = lax.axis_index("s")
    r0 = sid * ROWS_PER_SUB
    bufs = (rows_a, rows_b)
    gsems = (gsem0, gsem1)

    def zero_rows(i, c):
        for j in range(DW // 16):
            rows_a[i, pl.ds(j * 16, 16)] = jnp.zeros((16,), jnp.float32)
        return c
    lax.fori_loop(0, CHUNK, zero_rows, None)
    for t in range(ROWS_PER_SUB // CHUNK):
        pltpu.sync_copy(rows_a, acc_sh.at[pl.ds(r0 + t * CHUNK, CHUNK)])
    plsc.subcore_barrier()

    for h in range(CHUNKS // IDX_STAGE):
        pltpu.sync_copy(sidx_ref.at[sid, pl.ds(h * IDX_STAGE, IDX_STAGE)],
                        sidx_v)
        pltpu.sync_copy(didx_ref.at[sid, pl.ds(h * IDX_STAGE, IDX_STAGE)],
                        didx_v)
        g = {0: pltpu.async_copy(x_ref.at[sidx_v.at[0]], bufs[0], gsems[0])}
        sr = {}
        for j in range(IDX_STAGE):
            if j >= 1:
                sr[j - 1].wait()
            if j + 1 < IDX_STAGE:
                g[j + 1] = pltpu.async_copy(x_ref.at[sidx_v.at[j + 1]],
                                            bufs[(j + 1) % 2],
                                            gsems[(j + 1) % 2])
            g[j].wait()
            sr[j] = pltpu.async_copy(bufs[j % 2], acc_sh.at[didx_v.at[j]],
                                     srsem, add=True)
        sr[IDX_STAGE - 1].wait()
    plsc.subcore_barrier()

    for t in range(ROWS_PER_SUB // CHUNK):
        pltpu.sync_copy(acc_sh.at[pl.ds(r0 + t * CHUNK, CHUNK)], rows_a)
        pltpu.sync_copy(rows_a, sum_ref.at[pl.ds(r0 + t * CHUNK, CHUNK)])


def _sc_kernel(x_gene, x_cell, sidx_cg, didx_cg, sidx_gc, didx_gc,
               sum_g, sum_c,
               sidx_v, didx_v, rows_a, rows_b, acc_sh,
               gsem0, gsem1, srsem):
    cid = lax.axis_index("c")

    @pl.when(cid == 0)
    def _():
        _sc_aggregate_one(x_cell, sidx_cg, didx_cg, sum_g,
                          sidx_v, didx_v, rows_a, rows_b,
                          acc_sh, gsem0, gsem1, srsem)

    @pl.when(cid == 1)
    def _():
        _sc_aggregate_one(x_gene, sidx_gc, didx_gc, sum_c,
                          sidx_v, didx_v, rows_a, rows_b,
                          acc_sh, gsem0, gsem1, srsem)


def _sc_aggregate(x_gene_w, x_cell_w, ei_c2g, ei_g2c):
    sidx_cg, didx_cg = _edge_prep(ei_c2g)
    sidx_gc, didx_gc = _edge_prep(ei_g2c)
    mesh = plsc.VectorSubcoreMesh(core_axis_name="c", subcore_axis_name="s")
    out_type = [
        jax.ShapeDtypeStruct((ACC_ROWS, DW), jnp.float32),
        jax.ShapeDtypeStruct((ACC_ROWS, DW), jnp.float32),
    ]
    scratch = [
        pltpu.VMEM((IDX_STAGE, CHUNK), jnp.int32),
        pltpu.VMEM((IDX_STAGE, CHUNK), jnp.int32),
        pltpu.VMEM((CHUNK, DW), jnp.float32),
        pltpu.VMEM((CHUNK, DW), jnp.float32),
        pltpu.VMEM_SHARED((ACC_ROWS, DW), jnp.float32),
        pltpu.SemaphoreType.DMA,
        pltpu.SemaphoreType.DMA,
        pltpu.SemaphoreType.DMA,
    ]
    return pl.kernel(_sc_kernel, out_type=out_type, mesh=mesh,
                     scratch_types=scratch,
                     compiler_params=pltpu.CompilerParams(
                         use_tc_tiling_on_sc=False))(
        x_gene_w, x_cell_w, sidx_cg, didx_cg, sidx_gc, didx_gc)


BLK = 1000


def _tc_combine_kernel(sum_ref, x_ref, m_ref, wn_ref, ws_ref, b_ref,
                       out_ref):
    cnt = jnp.maximum(sum_ref[:, D:D + 1], 1.0)
    mean = sum_ref[:, 0:D] / cnt
    out = jnp.dot(mean, wn_ref[...], preferred_element_type=jnp.float32)
    out = out + jnp.dot(x_ref[...], ws_ref[...],
                        preferred_element_type=jnp.float32)
    out = out + b_ref[...]
    m = m_ref[...]
    out_ref[...] = out * (m + (1.0 - m) * 0.1)


def _tc_combine(summed, x_dst, m, w_n, w_s, b):
    nb = N // BLK
    return pl.pallas_call(
        _tc_combine_kernel,
        grid=(nb,),
        in_specs=[
            pl.BlockSpec((BLK, DW), lambda i: (i, 0)),
            pl.BlockSpec((BLK, D), lambda i: (i, 0)),
            pl.BlockSpec((BLK, 1), lambda i: (i, 0)),
            pl.BlockSpec((D, D), lambda i: (0, 0)),
            pl.BlockSpec((D, D), lambda i: (0, 0)),
            pl.BlockSpec((1, D), lambda i: (0, 0)),
        ],
        out_specs=pl.BlockSpec((BLK, D), lambda i: (i, 0)),
        out_shape=jax.ShapeDtypeStruct((N, D), jnp.float32),
    )(summed, x_dst, m, w_n, w_s, b)


def kernel(x_gene, x_cell, gene_mask, W_cg_n, W_cg_s, b_cg,
           W_gc_n, W_gc_s, b_gc, ei_c2g, ei_g2c):
    sum_g, sum_c = _sc_aggregate(_widen(x_gene), _widen(x_cell),
                                 ei_c2g, ei_g2c)
    b_cg2 = b_cg.reshape(1, D)
    b_gc2 = b_gc.reshape(1, D)
    m_gene = gene_mask.reshape(N, 1)
    m_one = jnp.ones((N, 1), jnp.float32)
    out_gene = _tc_combine(sum_g, x_gene, m_gene, W_cg_n, W_cg_s, b_cg2)
    out_cell = _tc_combine(sum_c, x_cell, m_one, W_gc_n, W_gc_s, b_gc2)
    return (out_gene, out_cell)


# speedup vs baseline: 1.5690x; 1.5690x over previous
import functools

import jax
import jax.numpy as jnp
from jax import lax
from jax.experimental import pallas as pl
from jax.experimental.pallas import tpu as pltpu
from jax.experimental.pallas import tpu_sc as plsc

N = 10000
E = 160000
D = 128

NS = 16
CHUNK = 128
CHUNKS = 80
EPAD = NS * CHUNKS * CHUNK
ROWS_PER_SUB = 640
ACC_ROWS = NS * ROWS_PER_SUB
DUMMY = N
CW = 16
IDX_STAGE = 10
NBUF = 4


def _edge_prep(ei):
    pad = EPAD - E
    src = jnp.concatenate([ei[0], jnp.zeros((pad,), jnp.int32)])
    dst = jnp.concatenate([ei[1], jnp.full((pad,), DUMMY, jnp.int32)])
    return src.reshape(NS, CHUNKS, CHUNK), dst.reshape(NS, CHUNKS, CHUNK)


def _sc_aggregate_one(x_ref, sidx_ref, didx_ref, sum_ref, cnt_ref,
                      sidx_v, didx_v, bufs, ones_v, acc_sh, cnt_sh,
                      gsems, srsem, scsem):
    sid = lax.axis_index("s")
    r0 = sid * ROWS_PER_SUB

    def zero_rows(i, c):
        for j in range(D // 32):
            bufs[0][i, pl.ds(j * 32, 32)] = jnp.zeros((32,), jnp.bfloat16)
        return c
    lax.fori_loop(0, CHUNK, zero_rows, None)

    def zero_cw(i, c):
        ones_v[i, :] = jnp.zeros((CW,), jnp.float32)
        return c
    lax.fori_loop(0, CHUNK, zero_cw, None)

    for t in range(ROWS_PER_SUB // CHUNK):
        pltpu.sync_copy(bufs[0], acc_sh.at[pl.ds(r0 + t * CHUNK, CHUNK)])
        pltpu.sync_copy(ones_v, cnt_sh.at[pl.ds(r0 + t * CHUNK, CHUNK)])

    def set_ones(i, c):
        ones_v[i, :] = jnp.ones((CW,), jnp.float32)
        return c
    lax.fori_loop(0, CHUNK, set_ones, None)
    plsc.subcore_barrier()

    for h in range(CHUNKS // IDX_STAGE):
        pltpu.sync_copy(sidx_ref.at[sid, pl.ds(h * IDX_STAGE, IDX_STAGE)],
                        sidx_v)
        pltpu.sync_copy(didx_ref.at[sid, pl.ds(h * IDX_STAGE, IDX_STAGE)],
                        didx_v)
        g = {}
        sr = {}
        sc = {}
        for j in range(NBUF - 1):
            g[j] = pltpu.async_copy(x_ref.at[sidx_v.at[j]], bufs[j % NBUF],
                                    gsems[j % NBUF])
        for j in range(IDX_STAGE):
            if j >= 1:
                sr[j - 1].wait()
                sc[j - 1].wait()
            if j + NBUF - 1 < IDX_STAGE:
                jj = j + NBUF - 1
                g[jj] = pltpu.async_copy(x_ref.at[sidx_v.at[jj]],
                                         bufs[jj % NBUF], gsems[jj % NBUF])
            g[j].wait()
            sr[j] = pltpu.async_copy(bufs[j % NBUF], acc_sh.at[didx_v.at[j]],
                                     srsem, add=True)
            sc[j] = pltpu.async_copy(ones_v, cnt_sh.at[didx_v.at[j]],
                                     scsem, add=True)
        sr[IDX_STAGE - 1].wait()
        sc[IDX_STAGE - 1].wait()
    plsc.subcore_barrier()

    for t in range(ROWS_PER_SUB // CHUNK):
        pltpu.sync_copy(acc_sh.at[pl.ds(r0 + t * CHUNK, CHUNK)], bufs[0])
        pltpu.sync_copy(bufs[0], sum_ref.at[pl.ds(r0 + t * CHUNK, CHUNK)])
        pltpu.sync_copy(cnt_sh.at[pl.ds(r0 + t * CHUNK, CHUNK)], ones_v)
        pltpu.sync_copy(ones_v, cnt_ref.at[pl.ds(r0 + t * CHUNK, CHUNK)])


def _sc_kernel(x_gene, x_cell, sidx_cg, didx_cg, sidx_gc, didx_gc,
               sum_g, cnt_g, sum_c, cnt_c,
               sidx_v, didx_v, rows_a, rows_b, rows_c, rows_d, ones_v,
               acc_sh, cnt_sh, gsem0, gsem1, gsem2, gsem3, srsem, scsem):
    cid = lax.axis_index("c")
    bufs = (rows_a, rows_b, rows_c, rows_d)
    gsems = (gsem0, gsem1, gsem2, gsem3)

    @pl.when(cid == 0)
    def _():
        _sc_aggregate_one(x_cell, sidx_cg, didx_cg, sum_g, cnt_g,
                          sidx_v, didx_v, bufs, ones_v, acc_sh, cnt_sh,
                          gsems, srsem, scsem)

    @pl.when(cid == 1)
    def _():
        _sc_aggregate_one(x_gene, sidx_gc, didx_gc, sum_c, cnt_c,
                          sidx_v, didx_v, bufs, ones_v, acc_sh, cnt_sh,
                          gsems, srsem, scsem)


def _sc_aggregate(x_gene_h, x_cell_h, ei_c2g, ei_g2c):
    sidx_cg, didx_cg = _edge_prep(ei_c2g)
    sidx_gc, didx_gc = _edge_prep(ei_g2c)
    mesh = plsc.VectorSubcoreMesh(core_axis_name="c", subcore_axis_name="s")
    out_type = [
        jax.ShapeDtypeStruct((ACC_ROWS, D), jnp.bfloat16),
        jax.ShapeDtypeStruct((ACC_ROWS, CW), jnp.float32),
        jax.ShapeDtypeStruct((ACC_ROWS, D), jnp.bfloat16),
        jax.ShapeDtypeStruct((ACC_ROWS, CW), jnp.float32),
    ]
    scratch = [
        pltpu.VMEM((IDX_STAGE, CHUNK), jnp.int32),
        pltpu.VMEM((IDX_STAGE, CHUNK), jnp.int32),
        pltpu.VMEM((CHUNK, D), jnp.bfloat16),
        pltpu.VMEM((CHUNK, D), jnp.bfloat16),
        pltpu.VMEM((CHUNK, D), jnp.bfloat16),
        pltpu.VMEM((CHUNK, D), jnp.bfloat16),
        pltpu.VMEM((CHUNK, CW), jnp.float32),
        pltpu.VMEM_SHARED((ACC_ROWS, D), jnp.bfloat16),
        pltpu.VMEM_SHARED((ACC_ROWS, CW), jnp.float32),
        pltpu.SemaphoreType.DMA,
        pltpu.SemaphoreType.DMA,
        pltpu.SemaphoreType.DMA,
        pltpu.SemaphoreType.DMA,
        pltpu.SemaphoreType.DMA,
        pltpu.SemaphoreType.DMA,
    ]
    return pl.kernel(_sc_kernel, out_type=out_type, mesh=mesh,
                     scratch_types=scratch,
                     compiler_params=pltpu.CompilerParams(
                         use_tc_tiling_on_sc=False))(
        x_gene_h, x_cell_h, sidx_cg, didx_cg, sidx_gc, didx_gc)


BLK = 1000


def _tc_combine_kernel(sum_ref, cnt_ref, x_ref, m_ref, wn_ref, ws_ref, b_ref,
                       out_ref):
    cnt = jnp.maximum(cnt_ref[:, 0:1], 1.0)
    mean = sum_ref[...].astype(jnp.float32) / cnt
    out = jnp.dot(mean, wn_ref[...], preferred_element_type=jnp.float32)
    out = out + jnp.dot(x_ref[...], ws_ref[...],
                        preferred_element_type=jnp.float32)
    out = out + b_ref[...]
    m = m_ref[...]
    out_ref[...] = out * (m + (1.0 - m) * 0.1)


def _tc_combine(summed, cnt, x_dst, m, w_n, w_s, b):
    nb = N // BLK
    return pl.pallas_call(
        _tc_combine_kernel,
        grid=(nb,),
        in_specs=[
            pl.BlockSpec((BLK, D), lambda i: (i, 0)),
            pl.BlockSpec((BLK, CW), lambda i: (i, 0)),
            pl.BlockSpec((BLK, D), lambda i: (i, 0)),
            pl.BlockSpec((BLK, 1), lambda i: (i, 0)),
            pl.BlockSpec((D, D), lambda i: (0, 0)),
            pl.BlockSpec((D, D), lambda i: (0, 0)),
            pl.BlockSpec((1, D), lambda i: (0, 0)),
        ],
        out_specs=pl.BlockSpec((BLK, D), lambda i: (i, 0)),
        out_shape=jax.ShapeDtypeStruct((N, D), jnp.float32),
    )(summed, cnt, x_dst, m, w_n, w_s, b)


def kernel(x_gene, x_cell, gene_mask, W_cg_n, W_cg_s, b_cg,
           W_gc_n, W_gc_s, b_gc, ei_c2g, ei_g2c):
    sum_g, cnt_g, sum_c, cnt_c = _sc_aggregate(
        x_gene.astype(jnp.bfloat16), x_cell.astype(jnp.bfloat16),
        ei_c2g, ei_g2c)
    b_cg2 = b_cg.reshape(1, D)
    b_gc2 = b_gc.reshape(1, D)
    m_gene = gene_mask.reshape(N, 1)
    m_one = jnp.ones((N, 1), jnp.float32)
    out_gene = _tc_combine(sum_g, cnt_g, x_gene, m_gene,
                           W_cg_n, W_cg_s, b_cg2)
    out_cell = _tc_combine(sum_c, cnt_c, x_cell, m_one,
                           W_gc_n, W_gc_s, b_gc2)
    return (out_gene, out_cell)


# speedup vs baseline: 1.8987x; 1.2102x over previous
import functools

import jax
import jax.numpy as jnp
from jax import lax
from jax.experimental import pallas as pl
from jax.experimental.pallas import tpu as pltpu
from jax.experimental.pallas import tpu_sc as plsc

N = 10000
E = 160000
D = 128

NS = 16
CHUNK = 128
CHUNKS = 80
EPAD = NS * CHUNKS * CHUNK
ROWS_PER_SUB = 640
ACC_ROWS = NS * ROWS_PER_SUB
DUMMY = N
CW = 16
IDX_STAGE = 10
NBUF = 4


def _edge_prep(ei):
    pad = EPAD - E
    src = jnp.concatenate([ei[0], jnp.zeros((pad,), jnp.int32)])
    dst = jnp.concatenate([ei[1], jnp.full((pad,), DUMMY, jnp.int32)])
    return src.reshape(NS, CHUNKS, CHUNK), dst.reshape(NS, CHUNKS, CHUNK)


def _sc_aggregate_one(x_ref, sidx_ref, didx_ref, sum_ref, cnt_ref,
                      sidx_v, didx_v, bufs, ones_v, acc_sh, cnt_sh, x_sh,
                      gsems, srsem, scsem):
    sid = lax.axis_index("s")
    r0 = sid * ROWS_PER_SUB

    for t in range(ROWS_PER_SUB // CHUNK):
        pltpu.sync_copy(x_ref.at[pl.ds(r0 + t * CHUNK, CHUNK)], bufs[1])
        pltpu.sync_copy(bufs[1], x_sh.at[pl.ds(r0 + t * CHUNK, CHUNK)])

    def zero_rows(i, c):
        for j in range(D // 32):
            bufs[0][i, pl.ds(j * 32, 32)] = jnp.zeros((32,), jnp.bfloat16)
        return c
    lax.fori_loop(0, CHUNK, zero_rows, None)

    def zero_cw(i, c):
        ones_v[i, :] = jnp.zeros((CW,), jnp.float32)
        return c
    lax.fori_loop(0, CHUNK, zero_cw, None)

    for t in range(ROWS_PER_SUB // CHUNK):
        pltpu.sync_copy(bufs[0], acc_sh.at[pl.ds(r0 + t * CHUNK, CHUNK)])
        pltpu.sync_copy(ones_v, cnt_sh.at[pl.ds(r0 + t * CHUNK, CHUNK)])

    def set_ones(i, c):
        ones_v[i, :] = jnp.ones((CW,), jnp.float32)
        return c
    lax.fori_loop(0, CHUNK, set_ones, None)
    plsc.subcore_barrier()

    for h in range(CHUNKS // IDX_STAGE):
        pltpu.sync_copy(sidx_ref.at[sid, pl.ds(h * IDX_STAGE, IDX_STAGE)],
                        sidx_v)
        pltpu.sync_copy(didx_ref.at[sid, pl.ds(h * IDX_STAGE, IDX_STAGE)],
                        didx_v)
        g = {}
        sr = {}
        sc = {}
        for j in range(NBUF - 1):
            g[j] = pltpu.async_copy(x_sh.at[sidx_v.at[j]], bufs[j % NBUF],
                                    gsems[j % NBUF])
        for j in range(IDX_STAGE):
            if j >= 1:
                sr[j - 1].wait()
                sc[j - 1].wait()
            if j + NBUF - 1 < IDX_STAGE:
                jj = j + NBUF - 1
                g[jj] = pltpu.async_copy(x_sh.at[sidx_v.at[jj]],
                                         bufs[jj % NBUF], gsems[jj % NBUF])
            g[j].wait()
            sr[j] = pltpu.async_copy(bufs[j % NBUF], acc_sh.at[didx_v.at[j]],
                                     srsem, add=True)
            sc[j] = pltpu.async_copy(ones_v, cnt_sh.at[didx_v.at[j]],
                                     scsem, add=True)
        sr[IDX_STAGE - 1].wait()
        sc[IDX_STAGE - 1].wait()
    plsc.subcore_barrier()

    for t in range(ROWS_PER_SUB // CHUNK):
        pltpu.sync_copy(acc_sh.at[pl.ds(r0 + t * CHUNK, CHUNK)], bufs[0])
        pltpu.sync_copy(bufs[0], sum_ref.at[pl.ds(r0 + t * CHUNK, CHUNK)])
        pltpu.sync_copy(cnt_sh.at[pl.ds(r0 + t * CHUNK, CHUNK)], ones_v)
        pltpu.sync_copy(ones_v, cnt_ref.at[pl.ds(r0 + t * CHUNK, CHUNK)])


def _sc_kernel(x_gene, x_cell, sidx_cg, didx_cg, sidx_gc, didx_gc,
               sum_g, cnt_g, sum_c, cnt_c,
               sidx_v, didx_v, rows_a, rows_b, rows_c, rows_d, ones_v,
               acc_sh, cnt_sh, x_sh, gsem0, gsem1, gsem2, gsem3,
               srsem, scsem):
    cid = lax.axis_index("c")
    bufs = (rows_a, rows_b, rows_c, rows_d)
    gsems = (gsem0, gsem1, gsem2, gsem3)

    @pl.when(cid == 0)
    def _():
        _sc_aggregate_one(x_cell, sidx_cg, didx_cg, sum_g, cnt_g,
                          sidx_v, didx_v, bufs, ones_v, acc_sh, cnt_sh,
                          x_sh, gsems, srsem, scsem)

    @pl.when(cid == 1)
    def _():
        _sc_aggregate_one(x_gene, sidx_gc, didx_gc, sum_c, cnt_c,
                          sidx_v, didx_v, bufs, ones_v, acc_sh, cnt_sh,
                          x_sh, gsems, srsem, scsem)


def _sc_aggregate(x_gene_h, x_cell_h, ei_c2g, ei_g2c):
    sidx_cg, didx_cg = _edge_prep(ei_c2g)
    sidx_gc, didx_gc = _edge_prep(ei_g2c)
    mesh = plsc.VectorSubcoreMesh(core_axis_name="c", subcore_axis_name="s")
    out_type = [
        jax.ShapeDtypeStruct((ACC_ROWS, D), jnp.bfloat16),
        jax.ShapeDtypeStruct((ACC_ROWS, CW), jnp.float32),
        jax.ShapeDtypeStruct((ACC_ROWS, D), jnp.bfloat16),
        jax.ShapeDtypeStruct((ACC_ROWS, CW), jnp.float32),
    ]
    scratch = [
        pltpu.VMEM((IDX_STAGE, CHUNK), jnp.int32),
        pltpu.VMEM((IDX_STAGE, CHUNK), jnp.int32),
        pltpu.VMEM((CHUNK, D), jnp.bfloat16),
        pltpu.VMEM((CHUNK, D), jnp.bfloat16),
        pltpu.VMEM((CHUNK, D), jnp.bfloat16),
        pltpu.VMEM((CHUNK, D), jnp.bfloat16),
        pltpu.VMEM((CHUNK, CW), jnp.float32),
        pltpu.VMEM_SHARED((ACC_ROWS, D), jnp.bfloat16),
        pltpu.VMEM_SHARED((ACC_ROWS, CW), jnp.float32),
        pltpu.VMEM_SHARED((ACC_ROWS, D), jnp.bfloat16),
        pltpu.SemaphoreType.DMA,
        pltpu.SemaphoreType.DMA,
        pltpu.SemaphoreType.DMA,
        pltpu.SemaphoreType.DMA,
        pltpu.SemaphoreType.DMA,
        pltpu.SemaphoreType.DMA,
    ]
    return pl.kernel(_sc_kernel, out_type=out_type, mesh=mesh,
                     scratch_types=scratch,
                     compiler_params=pltpu.CompilerParams(
                         use_tc_tiling_on_sc=False))(
        x_gene_h, x_cell_h, sidx_cg, didx_cg, sidx_gc, didx_gc)


BLK = 1000


def _tc_combine_kernel(sum_ref, cnt_ref, x_ref, m_ref, wn_ref, ws_ref, b_ref,
                       out_ref):
    cnt = jnp.maximum(cnt_ref[:, 0:1], 1.0)
    mean = sum_ref[...].astype(jnp.float32) / cnt
    out = jnp.dot(mean, wn_ref[...], preferred_element_type=jnp.float32)
    out = out + jnp.dot(x_ref[...], ws_ref[...],
                        preferred_element_type=jnp.float32)
    out = out + b_ref[...]
    m = m_ref[...]
    out_ref[...] = out * (m + (1.0 - m) * 0.1)


def _tc_combine(summed, cnt, x_dst, m, w_n, w_s, b):
    nb = N // BLK
    return pl.pallas_call(
        _tc_combine_kernel,
        grid=(nb,),
        in_specs=[
            pl.BlockSpec((BLK, D), lambda i: (i, 0)),
            pl.BlockSpec((BLK, CW), lambda i: (i, 0)),
            pl.BlockSpec((BLK, D), lambda i: (i, 0)),
            pl.BlockSpec((BLK, 1), lambda i: (i, 0)),
            pl.BlockSpec((D, D), lambda i: (0, 0)),
            pl.BlockSpec((D, D), lambda i: (0, 0)),
            pl.BlockSpec((1, D), lambda i: (0, 0)),
        ],
        out_specs=pl.BlockSpec((BLK, D), lambda i: (i, 0)),
        out_shape=jax.ShapeDtypeStruct((N, D), jnp.float32),
    )(summed, cnt, x_dst, m, w_n, w_s, b)


def kernel(x_gene, x_cell, gene_mask, W_cg_n, W_cg_s, b_cg,
           W_gc_n, W_gc_s, b_gc, ei_c2g, ei_g2c):
    zpad = jnp.zeros((ACC_ROWS - N, D), jnp.bfloat16)
    xg_h = jnp.concatenate([x_gene.astype(jnp.bfloat16), zpad])
    xc_h = jnp.concatenate([x_cell.astype(jnp.bfloat16), zpad])
    sum_g, cnt_g, sum_c, cnt_c = _sc_aggregate(xg_h, xc_h, ei_c2g, ei_g2c)
    b_cg2 = b_cg.reshape(1, D)
    b_gc2 = b_gc.reshape(1, D)
    m_gene = gene_mask.reshape(N, 1)
    m_one = jnp.ones((N, 1), jnp.float32)
    out_gene = _tc_combine(sum_g, cnt_g, x_gene, m_gene,
                           W_cg_n, W_cg_s, b_cg2)
    out_cell = _tc_combine(sum_c, cnt_c, x_cell, m_one,
                           W_gc_n, W_gc_s, b_gc2)
    return (out_gene, out_cell)


# speedup vs baseline: 2.0163x; 1.0619x over previous
import functools

import jax
import jax.numpy as jnp
from jax import lax
from jax.experimental import pallas as pl
from jax.experimental.pallas import tpu as pltpu
from jax.experimental.pallas import tpu_sc as plsc

N = 10000
E = 160000
D = 128

NS = 16
CHUNK = 128
CROWS = E // CHUNK
CHUNKS = CROWS // NS
XTRA = CROWS - NS * CHUNKS
ROWS_PER_SUB = 640
ACC_ROWS = NS * ROWS_PER_SUB
CW = 16
IDX_STAGE = 13
NBUF = 4


def _sc_aggregate_one(x_ref, e_ref, sum_ref, cnt_ref,
                      sidx_v, didx_v, bufs, ones_v, acc_sh, cnt_sh, x_sh,
                      gsems, srsem, scsem):
    sid = lax.axis_index("s")
    r0 = sid * ROWS_PER_SUB

    for t in range(ROWS_PER_SUB // CHUNK):
        pltpu.sync_copy(x_ref.at[pl.ds(r0 + t * CHUNK, CHUNK)], bufs[1])
        pltpu.sync_copy(bufs[1], x_sh.at[pl.ds(r0 + t * CHUNK, CHUNK)])

    def zero_rows(i, c):
        for j in range(D // 32):
            bufs[0][i, pl.ds(j * 32, 32)] = jnp.zeros((32,), jnp.bfloat16)
        return c
    lax.fori_loop(0, CHUNK, zero_rows, None)

    def zero_cw(i, c):
        ones_v[i, :] = jnp.zeros((CW,), jnp.float32)
        return c
    lax.fori_loop(0, CHUNK, zero_cw, None)

    for t in range(ROWS_PER_SUB // CHUNK):
        pltpu.sync_copy(bufs[0], acc_sh.at[pl.ds(r0 + t * CHUNK, CHUNK)])
        pltpu.sync_copy(ones_v, cnt_sh.at[pl.ds(r0 + t * CHUNK, CHUNK)])

    def set_ones(i, c):
        ones_v[i, :] = jnp.ones((CW,), jnp.float32)
        return c
    lax.fori_loop(0, CHUNK, set_ones, None)
    plsc.subcore_barrier()

    c0 = sid * CHUNKS
    for h in range(CHUNKS // IDX_STAGE):
        pltpu.sync_copy(e_ref.at[0, pl.ds(c0 + h * IDX_STAGE, IDX_STAGE)],
                        sidx_v)
        pltpu.sync_copy(e_ref.at[1, pl.ds(c0 + h * IDX_STAGE, IDX_STAGE)],
                        didx_v)
        g = {}
        sr = {}
        sc = {}
        for j in range(NBUF - 1):
            g[j] = pltpu.async_copy(x_sh.at[sidx_v.at[j]], bufs[j % NBUF],
                                    gsems[j % NBUF])
        for j in range(IDX_STAGE):
            if j >= 1:
                sr[j - 1].wait()
                sc[j - 1].wait()
            if j + NBUF - 1 < IDX_STAGE:
                jj = j + NBUF - 1
                g[jj] = pltpu.async_copy(x_sh.at[sidx_v.at[jj]],
                                         bufs[jj % NBUF], gsems[jj % NBUF])
            g[j].wait()
            sr[j] = pltpu.async_copy(bufs[j % NBUF], acc_sh.at[didx_v.at[j]],
                                     srsem, add=True)
            sc[j] = pltpu.async_copy(ones_v, cnt_sh.at[didx_v.at[j]],
                                     scsem, add=True)
        sr[IDX_STAGE - 1].wait()
        sc[IDX_STAGE - 1].wait()

    @pl.when(sid < XTRA)
    def _():
        pltpu.sync_copy(e_ref.at[0, pl.ds(NS * CHUNKS + sid, 1)],
                        sidx_v.at[pl.ds(0, 1)])
        pltpu.sync_copy(e_ref.at[1, pl.ds(NS * CHUNKS + sid, 1)],
                        didx_v.at[pl.ds(0, 1)])
        g = pltpu.async_copy(x_sh.at[sidx_v.at[0]], bufs[0], gsems[0])
        g.wait()
        sr = pltpu.async_copy(bufs[0], acc_sh.at[didx_v.at[0]],
                              srsem, add=True)
        sc = pltpu.async_copy(ones_v, cnt_sh.at[didx_v.at[0]],
                              scsem, add=True)
        sr.wait()
        sc.wait()
    plsc.subcore_barrier()

    for t in range(ROWS_PER_SUB // CHUNK):
        pltpu.sync_copy(acc_sh.at[pl.ds(r0 + t * CHUNK, CHUNK)], bufs[0])
        pltpu.sync_copy(bufs[0], sum_ref.at[pl.ds(r0 + t * CHUNK, CHUNK)])
        pltpu.sync_copy(cnt_sh.at[pl.ds(r0 + t * CHUNK, CHUNK)], ones_v)
        pltpu.sync_copy(ones_v, cnt_ref.at[pl.ds(r0 + t * CHUNK, CHUNK)])


def _sc_kernel(x_gene, x_cell, e_cg, e_gc,
               sum_g, cnt_g, sum_c, cnt_c,
               sidx_v, didx_v, rows_a, rows_b, rows_c, rows_d, ones_v,
               acc_sh, cnt_sh, x_sh, gsem0, gsem1, gsem2, gsem3,
               srsem, scsem):
    cid = lax.axis_index("c")
    bufs = (rows_a, rows_b, rows_c, rows_d)
    gsems = (gsem0, gsem1, gsem2, gsem3)

    @pl.when(cid == 0)
    def _():
        _sc_aggregate_one(x_cell, e_cg, sum_g, cnt_g,
                          sidx_v, didx_v, bufs, ones_v, acc_sh, cnt_sh,
                          x_sh, gsems, srsem, scsem)

    @pl.when(cid == 1)
    def _():
        _sc_aggregate_one(x_gene, e_gc, sum_c, cnt_c,
                          sidx_v, didx_v, bufs, ones_v, acc_sh, cnt_sh,
                          x_sh, gsems, srsem, scsem)


def _sc_aggregate(x_gene_h, x_cell_h, e_cg, e_gc):
    mesh = plsc.VectorSubcoreMesh(core_axis_name="c", subcore_axis_name="s")
    out_type = [
        jax.ShapeDtypeStruct((ACC_ROWS, D), jnp.bfloat16),
        jax.ShapeDtypeStruct((ACC_ROWS, CW), jnp.float32),
        jax.ShapeDtypeStruct((ACC_ROWS, D), jnp.bfloat16),
        jax.ShapeDtypeStruct((ACC_ROWS, CW), jnp.float32),
    ]
    scratch = [
        pltpu.VMEM((IDX_STAGE, CHUNK), jnp.int32),
        pltpu.VMEM((IDX_STAGE, CHUNK), jnp.int32),
        pltpu.VMEM((CHUNK, D), jnp.bfloat16),
        pltpu.VMEM((CHUNK, D), jnp.bfloat16),
        pltpu.VMEM((CHUNK, D), jnp.bfloat16),
        pltpu.VMEM((CHUNK, D), jnp.bfloat16),
        pltpu.VMEM((CHUNK, CW), jnp.float32),
        pltpu.VMEM_SHARED((ACC_ROWS, D), jnp.bfloat16),
        pltpu.VMEM_SHARED((ACC_ROWS, CW), jnp.float32),
        pltpu.VMEM_SHARED((ACC_ROWS, D), jnp.bfloat16),
        pltpu.SemaphoreType.DMA,
        pltpu.SemaphoreType.DMA,
        pltpu.SemaphoreType.DMA,
        pltpu.SemaphoreType.DMA,
        pltpu.SemaphoreType.DMA,
        pltpu.SemaphoreType.DMA,
    ]
    return pl.kernel(_sc_kernel, out_type=out_type, mesh=mesh,
                     scratch_types=scratch,
                     compiler_params=pltpu.CompilerParams(
                         use_tc_tiling_on_sc=False))(
        x_gene_h, x_cell_h, e_cg, e_gc)


BLK = 1000


def _tc_combine_kernel(sum_ref, cnt_ref, x_ref, m_ref, wn_ref, ws_ref, b_ref,
                       out_ref):
    cnt = jnp.maximum(cnt_ref[:, 0:1], 1.0)
    mean = sum_ref[...].astype(jnp.float32) / cnt
    out = jnp.dot(mean, wn_ref[...], preferred_element_type=jnp.float32)
    out = out + jnp.dot(x_ref[...], ws_ref[...],
                        preferred_element_type=jnp.float32)
    out = out + b_ref[...]
    m = m_ref[...]
    out_ref[...] = out * (m + (1.0 - m) * 0.1)


def _tc_combine(summed, cnt, x_dst, m, w_n, w_s, b):
    nb = N // BLK
    return pl.pallas_call(
        _tc_combine_kernel,
        grid=(nb,),
        in_specs=[
            pl.BlockSpec((BLK, D), lambda i: (i, 0)),
            pl.BlockSpec((BLK, CW), lambda i: (i, 0)),
            pl.BlockSpec((BLK, D), lambda i: (i, 0)),
            pl.BlockSpec((BLK, 1), lambda i: (i, 0)),
            pl.BlockSpec((D, D), lambda i: (0, 0)),
            pl.BlockSpec((D, D), lambda i: (0, 0)),
            pl.BlockSpec((1, D), lambda i: (0, 0)),
        ],
        out_specs=pl.BlockSpec((BLK, D), lambda i: (i, 0)),
        out_shape=jax.ShapeDtypeStruct((N, D), jnp.float32),
    )(summed, cnt, x_dst, m, w_n, w_s, b)


def kernel(x_gene, x_cell, gene_mask, W_cg_n, W_cg_s, b_cg,
           W_gc_n, W_gc_s, b_gc, ei_c2g, ei_g2c):
    zpad = jnp.zeros((ACC_ROWS - N, D), jnp.bfloat16)
    xg_h = jnp.concatenate([x_gene.astype(jnp.bfloat16), zpad])
    xc_h = jnp.concatenate([x_cell.astype(jnp.bfloat16), zpad])
    e_cg = ei_c2g.reshape(2, CROWS, CHUNK)
    e_gc = ei_g2c.reshape(2, CROWS, CHUNK)
    sum_g, cnt_g, sum_c, cnt_c = _sc_aggregate(xg_h, xc_h, e_cg, e_gc)
    b_cg2 = b_cg.reshape(1, D)
    b_gc2 = b_gc.reshape(1, D)
    m_gene = gene_mask.reshape(N, 1)
    m_one = jnp.ones((N, 1), jnp.float32)
    out_gene = _tc_combine(sum_g, cnt_g, x_gene, m_gene,
                           W_cg_n, W_cg_s, b_cg2)
    out_cell = _tc_combine(sum_c, cnt_c, x_cell, m_one,
                           W_gc_n, W_gc_s, b_gc2)
    return (out_gene, out_cell)


# speedup vs baseline: 2.0173x; 1.0005x over previous
import functools

import jax
import jax.numpy as jnp
from jax import lax
from jax.experimental import pallas as pl
from jax.experimental.pallas import tpu as pltpu
from jax.experimental.pallas import tpu_sc as plsc

N = 10000
E = 160000
D = 128

NS = 16
CHUNK = 128
CROWS = E // CHUNK
CHUNKS = CROWS // NS
XTRA = CROWS - NS * CHUNKS
ROWS_PER_SUB = 640
ACC_ROWS = NS * ROWS_PER_SUB
CW = 16
IDX_STAGE = 13
NBUF = 4


def _sc_aggregate_one(x_ref, e_ref, sum_ref, cnt_ref,
                      sidx_v, didx_v, bufs, ones_v, acc_sh, cnt_sh, x_sh,
                      gsems, srsem, scsem):
    sid = lax.axis_index("s")
    r0 = sid * ROWS_PER_SUB

    x0 = sid * (N // NS)
    for t in range(5):
        pltpu.sync_copy(x_ref.at[pl.ds(x0 + t * 125, 125)],
                        bufs[1].at[pl.ds(0, 125)])
        pltpu.sync_copy(bufs[1].at[pl.ds(0, 125)],
                        x_sh.at[pl.ds(x0 + t * 125, 125)])

    def zero_rows(i, c):
        for j in range(D // 32):
            bufs[0][i, pl.ds(j * 32, 32)] = jnp.zeros((32,), jnp.bfloat16)
        return c
    lax.fori_loop(0, CHUNK, zero_rows, None)

    def zero_cw(i, c):
        ones_v[i, :] = jnp.zeros((CW,), jnp.float32)
        return c
    lax.fori_loop(0, CHUNK, zero_cw, None)

    for t in range(ROWS_PER_SUB // CHUNK):
        pltpu.sync_copy(bufs[0], acc_sh.at[pl.ds(r0 + t * CHUNK, CHUNK)])
        pltpu.sync_copy(ones_v, cnt_sh.at[pl.ds(r0 + t * CHUNK, CHUNK)])

    def set_ones(i, c):
        ones_v[i, :] = jnp.ones((CW,), jnp.float32)
        return c
    lax.fori_loop(0, CHUNK, set_ones, None)
    plsc.subcore_barrier()

    c0 = sid * CHUNKS
    for h in range(CHUNKS // IDX_STAGE):
        pltpu.sync_copy(e_ref.at[0, pl.ds(c0 + h * IDX_STAGE, IDX_STAGE)],
                        sidx_v)
        pltpu.sync_copy(e_ref.at[1, pl.ds(c0 + h * IDX_STAGE, IDX_STAGE)],
                        didx_v)
        g = {}
        sr = {}
        sc = {}
        for j in range(NBUF - 1):
            g[j] = pltpu.async_copy(x_sh.at[sidx_v.at[j]], bufs[j % NBUF],
                                    gsems[j % NBUF])
        for j in range(IDX_STAGE):
            if j >= 1:
                sr[j - 1].wait()
                sc[j - 1].wait()
            if j + NBUF - 1 < IDX_STAGE:
                jj = j + NBUF - 1
                g[jj] = pltpu.async_copy(x_sh.at[sidx_v.at[jj]],
                                         bufs[jj % NBUF], gsems[jj % NBUF])
            g[j].wait()
            sr[j] = pltpu.async_copy(bufs[j % NBUF], acc_sh.at[didx_v.at[j]],
                                     srsem, add=True)
            sc[j] = pltpu.async_copy(ones_v, cnt_sh.at[didx_v.at[j]],
                                     scsem, add=True)
        sr[IDX_STAGE - 1].wait()
        sc[IDX_STAGE - 1].wait()

    @pl.when(sid < XTRA)
    def _():
        pltpu.sync_copy(e_ref.at[0, pl.ds(NS * CHUNKS + sid, 1)],
                        sidx_v.at[pl.ds(0, 1)])
        pltpu.sync_copy(e_ref.at[1, pl.ds(NS * CHUNKS + sid, 1)],
                        didx_v.at[pl.ds(0, 1)])
        g = pltpu.async_copy(x_sh.at[sidx_v.at[0]], bufs[0], gsems[0])
        g.wait()
        sr = pltpu.async_copy(bufs[0], acc_sh.at[didx_v.at[0]],
                              srsem, add=True)
        sc = pltpu.async_copy(ones_v, cnt_sh.at[didx_v.at[0]],
                              scsem, add=True)
        sr.wait()
        sc.wait()
    plsc.subcore_barrier()

    for t in range(ROWS_PER_SUB // CHUNK):
        pltpu.sync_copy(acc_sh.at[pl.ds(r0 + t * CHUNK, CHUNK)], bufs[0])
        pltpu.sync_copy(bufs[0], sum_ref.at[pl.ds(r0 + t * CHUNK, CHUNK)])
        pltpu.sync_copy(cnt_sh.at[pl.ds(r0 + t * CHUNK, CHUNK)], ones_v)
        pltpu.sync_copy(ones_v, cnt_ref.at[pl.ds(r0 + t * CHUNK, CHUNK)])


def _sc_kernel(x_gene, x_cell, e_cg, e_gc,
               sum_g, cnt_g, sum_c, cnt_c,
               sidx_v, didx_v, rows_a, rows_b, rows_c, rows_d, ones_v,
               acc_sh, cnt_sh, x_sh, gsem0, gsem1, gsem2, gsem3,
               srsem, scsem):
    cid = lax.axis_index("c")
    bufs = (rows_a, rows_b, rows_c, rows_d)
    gsems = (gsem0, gsem1, gsem2, gsem3)

    @pl.when(cid == 0)
    def _():
        _sc_aggregate_one(x_cell, e_cg, sum_g, cnt_g,
                          sidx_v, didx_v, bufs, ones_v, acc_sh, cnt_sh,
                          x_sh, gsems, srsem, scsem)

    @pl.when(cid == 1)
    def _():
        _sc_aggregate_one(x_gene, e_gc, sum_c, cnt_c,
                          sidx_v, didx_v, bufs, ones_v, acc_sh, cnt_sh,
                          x_sh, gsems, srsem, scsem)


def _sc_aggregate(x_gene_h, x_cell_h, e_cg, e_gc):
    mesh = plsc.VectorSubcoreMesh(core_axis_name="c", subcore_axis_name="s")
    out_type = [
        jax.ShapeDtypeStruct((ACC_ROWS, D), jnp.bfloat16),
        jax.ShapeDtypeStruct((ACC_ROWS, CW), jnp.float32),
        jax.ShapeDtypeStruct((ACC_ROWS, D), jnp.bfloat16),
        jax.ShapeDtypeStruct((ACC_ROWS, CW), jnp.float32),
    ]
    scratch = [
        pltpu.VMEM((IDX_STAGE, CHUNK), jnp.int32),
        pltpu.VMEM((IDX_STAGE, CHUNK), jnp.int32),
        pltpu.VMEM((CHUNK, D), jnp.bfloat16),
        pltpu.VMEM((CHUNK, D), jnp.bfloat16),
        pltpu.VMEM((CHUNK, D), jnp.bfloat16),
        pltpu.VMEM((CHUNK, D), jnp.bfloat16),
        pltpu.VMEM((CHUNK, CW), jnp.float32),
        pltpu.VMEM_SHARED((ACC_ROWS, D), jnp.bfloat16),
        pltpu.VMEM_SHARED((ACC_ROWS, CW), jnp.float32),
        pltpu.VMEM_SHARED((ACC_ROWS, D), jnp.bfloat16),
        pltpu.SemaphoreType.DMA,
        pltpu.SemaphoreType.DMA,
        pltpu.SemaphoreType.DMA,
        pltpu.SemaphoreType.DMA,
        pltpu.SemaphoreType.DMA,
        pltpu.SemaphoreType.DMA,
    ]
    return pl.kernel(_sc_kernel, out_type=out_type, mesh=mesh,
                     scratch_types=scratch,
                     compiler_params=pltpu.CompilerParams(
                         use_tc_tiling_on_sc=False))(
        x_gene_h, x_cell_h, e_cg, e_gc)


BLK = 1000


def _tc_combine_kernel(sum_ref, cnt_ref, x_ref, m_ref, wn_ref, ws_ref, b_ref,
                       out_ref):
    cnt = jnp.maximum(cnt_ref[:, 0:1], 1.0)
    mean = sum_ref[...].astype(jnp.float32) / cnt
    out = jnp.dot(mean, wn_ref[...], preferred_element_type=jnp.float32)
    out = out + jnp.dot(x_ref[...], ws_ref[...],
                        preferred_element_type=jnp.float32)
    out = out + b_ref[...]
    m = m_ref[...]
    out_ref[...] = out * (m + (1.0 - m) * 0.1)


def _tc_combine_plain_kernel(sum_ref, cnt_ref, x_ref, wn_ref, ws_ref, b_ref,
                             out_ref):
    cnt = jnp.maximum(cnt_ref[:, 0:1], 1.0)
    mean = sum_ref[...].astype(jnp.float32) / cnt
    out = jnp.dot(mean, wn_ref[...], preferred_element_type=jnp.float32)
    out = out + jnp.dot(x_ref[...], ws_ref[...],
                        preferred_element_type=jnp.float32)
    out_ref[...] = out + b_ref[...]


def _tc_combine(summed, cnt, x_dst, m, w_n, w_s, b):
    nb = N // BLK
    specs = [
        pl.BlockSpec((BLK, D), lambda i: (i, 0)),
        pl.BlockSpec((BLK, CW), lambda i: (i, 0)),
        pl.BlockSpec((BLK, D), lambda i: (i, 0)),
        pl.BlockSpec((BLK, 1), lambda i: (i, 0)),
        pl.BlockSpec((D, D), lambda i: (0, 0)),
        pl.BlockSpec((D, D), lambda i: (0, 0)),
        pl.BlockSpec((1, D), lambda i: (0, 0)),
    ]
    if m is None:
        return pl.pallas_call(
            _tc_combine_plain_kernel,
            grid=(nb,),
            in_specs=specs[:3] + specs[4:],
            out_specs=pl.BlockSpec((BLK, D), lambda i: (i, 0)),
            out_shape=jax.ShapeDtypeStruct((N, D), jnp.float32),
        )(summed, cnt, x_dst, w_n, w_s, b)
    return pl.pallas_call(
        _tc_combine_kernel,
        grid=(nb,),
        in_specs=specs,
        out_specs=pl.BlockSpec((BLK, D), lambda i: (i, 0)),
        out_shape=jax.ShapeDtypeStruct((N, D), jnp.float32),
    )(summed, cnt, x_dst, m, w_n, w_s, b)


def kernel(x_gene, x_cell, gene_mask, W_cg_n, W_cg_s, b_cg,
           W_gc_n, W_gc_s, b_gc, ei_c2g, ei_g2c):
    xg_h = x_gene.astype(jnp.bfloat16)
    xc_h = x_cell.astype(jnp.bfloat16)
    e_cg = ei_c2g.reshape(2, CROWS, CHUNK)
    e_gc = ei_g2c.reshape(2, CROWS, CHUNK)
    sum_g, cnt_g, sum_c, cnt_c = _sc_aggregate(xg_h, xc_h, e_cg, e_gc)
    b_cg2 = b_cg.reshape(1, D)
    b_gc2 = b_gc.reshape(1, D)
    m_gene = gene_mask.reshape(N, 1)
    out_gene = _tc_combine(sum_g, cnt_g, x_gene, m_gene,
                           W_cg_n, W_cg_s, b_cg2)
    out_cell = _tc_combine(sum_c, cnt_c, x_cell, None,
                           W_gc_n, W_gc_s, b_gc2)
    return (out_gene, out_cell)
